# baseline (speedup 1.0000x reference)
import functools

import jax
import jax.numpy as jnp
from jax import lax
from jax.experimental import pallas as pl
from jax.experimental.pallas import tpu as pltpu
from jax.experimental.pallas import tpu_sc as plsc

_C = 21
_BATCH = 8192
_NW = 32
_IJS = 52
_NCHUNK = _IJS // 4
_PW = 8 * 1024
_TW = 8 * 512
_SCALE = 6.5131 / 40.0
_LN2 = 0.6931471805599453


def _sc_loss_partials(pred_lin, tgt_lin):
  mesh = plsc.VectorSubcoreMesh(core_axis_name="c", subcore_axis_name="s")

  @functools.partial(
      pl.kernel,
      out_type=jax.ShapeDtypeStruct((_NW * 80,), jnp.float32),
      mesh=mesh,
      compiler_params=pltpu.CompilerParams(needs_layout_passes=False),
      scratch_types=[
          pltpu.VMEM((2 * 3 * _PW,), jnp.float32),
          pltpu.VMEM((2 * _TW,), jnp.float32),
          pltpu.VMEM((80,), jnp.float32),
          pltpu.SemaphoreType.DMA,
          pltpu.SemaphoreType.DMA,
      ],
  )
  def k(pred_hbm, tgt_hbm, out_hbm, pred_buf, tgt_buf, acc_buf, sem0, sem1):
    wid = lax.axis_index("s") * 2 + lax.axis_index("c")

    zero = jnp.zeros((16,), jnp.float32)
    one = jnp.ones((16,), jnp.float32)
    ninf = jnp.full((16,), -jnp.inf, jnp.float32)

    def sigmoid(x):
      return 1.0 / (1.0 + jnp.exp(-x))

    def log_pos(u):
      bits = lax.bitcast_convert_type(u, jnp.int32)
      ex = lax.shift_right_logical(bits, 23) - 127
      m = lax.bitcast_convert_type(
          lax.bitwise_or(lax.bitwise_and(bits, 0x007FFFFF), 0x3F800000),
          jnp.float32)
      s = (m - 1.0) / (m + 1.0)
      t = s * s
      poly = 1.0 + t * (1.0 / 3.0 + t * (0.2 + t * (1.0 / 7.0 + t * (1.0 / 9.0))))
      lg = ex.astype(jnp.float32) * _LN2 + 2.0 * s * poly
      return jnp.where(u > 0, lg, ninf)

    def make_group_body(slot):
     def group_body(g, accs):
      xy, wh, obj, noobj, cls = accs
      u = lax.shift_right_logical(g, 3)
      sub = lax.bitwise_and(g, 7)
      q = u * 1024 + sub * 16 + slot * (3 * _PW)
      r = u * 512 + sub * 16 + slot * _TW

      t0 = tgt_buf[pl.ds(r, 16)]
      t1 = tgt_buf[pl.ds(r + 128, 16)]
      t2 = tgt_buf[pl.ds(r + 256, 16)]
      t3 = tgt_buf[pl.ds(r + 384, 16)]
      p0 = pred_buf[pl.ds(q, 16)]
      p1 = pred_buf[pl.ds(q + 128, 16)]
      p2 = pred_buf[pl.ds(q + 256, 16)]

      cm = t2 > 0.0
      d2 = sigmoid(p2) - t2
      s2 = d2 * d2
      o = jnp.where(cm, s2, zero)
      obj = obj + o
      noobj = noobj + (s2 - o)
      d0 = sigmoid(p0) - t0
      xy = xy + jnp.where(cm, d0 * d0, zero)
      d1 = p1 - log_pos(t1 * (1.0 / _SCALE))
      wh = wh + jnp.where(cm, d1 * d1, zero)

      num = zero
      den = zero
      for c in range(_C):
        cc = 3 + c
        off = (cc // 8) * _PW + (cc % 8) * 128
        e = jnp.exp(pred_buf[pl.ds(q + off, 16)])
        den = den + e
        num = num + e * (1.0 + 0.5 * c)
      diff = (10.0 * (num + den)) / (den * (t3 + 1.0)) - 10.0
      ad = jnp.abs(diff)
      sl1 = jnp.where(ad < 1.0, 0.5 * diff * diff, ad - 0.5)
      cls = cls + jnp.where(jnp.logical_and(cm, t3 > 0.0), sl1, zero)
      return (xy, wh, obj, noobj, cls)
     return group_body

    group_bodies = (make_group_body(0), make_group_body(1))
    sems = (sem0, sem1)

    def chunk_copies(ck, slot):
      gc = wid * _NCHUNK + ck
      ij = lax.shift_right_logical(gc, 3)
      bt0 = lax.bitwise_and(gc, 7) * 8
      pbase = (ij * 192 + bt0) * 1024
      ds = [
          pltpu.make_async_copy(
              pred_hbm.at[pl.ds(pbase + ct * 64 * 1024, _PW)],
              pred_buf.at[pl.ds(slot * 3 * _PW + ct * _PW, _PW)],
              sems[slot])
          for ct in range(3)
      ]
      ds.append(pltpu.make_async_copy(
          tgt_hbm.at[pl.ds((ij * 64 + bt0) * 512, _TW)],
          tgt_buf.at[pl.ds(slot * _TW, _TW)],
          sems[slot]))
      return ds

    def start_chunk(ck, slot):
      for d in chunk_copies(ck, slot):
        d.start()

    def wait_chunk(ck, slot):
      for d in chunk_copies(ck, slot):
        d.wait()

    def pair_body(m, accs):
      c0 = 2 * m
      start_chunk(c0 + 1, 1)
      wait_chunk(c0, 0)
      accs = lax.fori_loop(0, 64, group_bodies[0], accs)
      start_chunk(c0 + 2, 0)
      wait_chunk(c0 + 1, 1)
      accs = lax.fori_loop(0, 64, group_bodies[1], accs)
      return accs

    accs = (zero, zero, zero, zero, zero)
    start_chunk(0, 0)
    accs = lax.fori_loop(0, (_NCHUNK - 1) // 2, pair_body, accs)
    wait_chunk(_NCHUNK - 1, 0)
    accs = lax.fori_loop(0, 64, group_bodies[0], accs)
    xy, wh, obj, noobj, cls = accs
    acc_buf[pl.ds(0, 16)] = xy
    acc_buf[pl.ds(16, 16)] = wh
    acc_buf[pl.ds(32, 16)] = obj
    acc_buf[pl.ds(48, 16)] = noobj
    acc_buf[pl.ds(64, 16)] = cls
    pltpu.sync_copy(acc_buf, out_hbm.at[pl.ds(wid * 80, 80)])

  return k(pred_lin, tgt_lin)


def _tc_loss_partials(pred5, tgt4):
  nij = 100 - _IJS

  def body(pred_ref, tgt_ref, out_ref):
    ci = lax.broadcasted_iota(jnp.int32, (1, 8, 1), 1).astype(jnp.float32)
    w1s, wms = [], []
    for ct in range(3):
      ch = ci + float(ct * 8)
      w1 = (ch >= 3.0).astype(jnp.float32)
      w1s.append(w1)
      wms.append(w1 * (0.5 * ch - 0.5))
    zero = jnp.zeros((8, 128), jnp.float32)
    accs = [zero] * 5

    for g in range(8):
      tg = tgt_ref[0, pl.ds(8 * g, 8)]
      t0 = tg[:, 0, :]
      t1 = tg[:, 1, :]
      t2 = tg[:, 2, :]
      t3 = tg[:, 3, :]
      x0 = pred_ref[0, 0, pl.ds(8 * g, 8)]
      x1 = pred_ref[0, 1, pl.ds(8 * g, 8)]
      x2 = pred_ref[0, 2, pl.ds(8 * g, 8)]
      p0 = x0[:, 0, :]
      p1 = x0[:, 1, :]
      p2 = x0[:, 2, :]

      cm = t2 > 0.0
      d2 = 1.0 / (1.0 + jnp.exp(-p2)) - t2
      s2 = d2 * d2
      o = jnp.where(cm, s2, zero)
      obj = o
      noobj = s2 - o
      d0 = 1.0 / (1.0 + jnp.exp(-p0)) - t0
      xy = jnp.where(cm, d0 * d0, zero)
      d1 = p1 - jnp.log(t1 * (1.0 / _SCALE))
      wh = jnp.where(cm, d1 * d1, zero)

      e0 = jnp.exp(x0)
      dacc = e0 * w1s[0]
      nacc = e0 * wms[0]
      e1 = jnp.exp(x1)
      dacc = dacc + e1
      nacc = nacc + e1 * wms[1]
      e2 = jnp.exp(x2)
      dacc = dacc + e2
      nacc = nacc + e2 * wms[2]
      den = dacc.sum(axis=1)
      num = nacc.sum(axis=1)
      diff = (10.0 * (num + den)) / (den * (t3 + 1.0)) - 10.0
      ad = jnp.abs(diff)
      sl1 = jnp.where(ad < 1.0, 0.5 * diff * diff, ad - 0.5)
      cls = jnp.where(jnp.logical_and(cm, t3 > 0.0), sl1, zero)

      for a, v in enumerate((xy, wh, obj, noobj, cls)):
        accs[a] = accs[a] + v

    acc = jnp.concatenate(accs, axis=0)
    @pl.when(pl.program_id(0) == 0)
    def _():
      out_ref[...] = acc
    @pl.when(pl.program_id(0) > 0)
    def _():
      out_ref[...] += acc

  return pl.pallas_call(
      body,
      grid=(nij,),
      in_specs=[
          pl.BlockSpec((1, 3, 64, 8, 128), lambda i: (_IJS + i, 0, 0, 0, 0)),
          pl.BlockSpec((1, 64, 4, 128), lambda i: (_IJS + i, 0, 0, 0)),
      ],
      out_specs=pl.BlockSpec((40, 128), lambda i: (0, 0)),
      out_shape=jax.ShapeDtypeStruct((40, 128), jnp.float32),
      compiler_params=pltpu.CompilerParams(
          dimension_semantics=("arbitrary",)),
  )(pred5, tgt4)


def _linear_view_pred(p):
  return (p.transpose(1, 2, 3, 0)
           .reshape(10, 10, 3, 8, 64, 128)
           .transpose(0, 1, 2, 4, 3, 5)
           .reshape(-1))


def _linear_view_tgt(t):
  return (t.transpose(1, 2, 3, 0)
           .reshape(10, 10, 1, 4, 64, 128)
           .transpose(0, 1, 2, 4, 3, 5)
           .reshape(-1))


def kernel(pred_tensor, target_tensor):
  pred_lin = _linear_view_pred(pred_tensor)
  tgt_lin = _linear_view_tgt(target_tensor)
  tc_parts = _tc_loss_partials(pred_lin.reshape(100, 3, 64, 8, 128),
                               tgt_lin.reshape(100, 64, 4, 128))
  parts = _sc_loss_partials(pred_lin, tgt_lin)
  sums = (parts.reshape(_NW, 5, 16).sum(axis=(0, 2))
          + tc_parts.reshape(5, 1024).sum(axis=1))
  xy, wh, obj, noobj, cls = sums[0], sums[1], sums[2], sums[3], sums[4]
  loss = 10.0 * (xy + wh) + obj + noobj + 0.5 * cls
  bs = jnp.float32(_BATCH)
  return (xy / bs, wh / bs, obj / bs, noobj / bs, cls / bs, loss / bs)


# speedup vs baseline: 1.0914x; 1.0914x over previous
import functools

import jax
import jax.numpy as jnp
from jax import lax
from jax.experimental import pallas as pl
from jax.experimental.pallas import tpu as pltpu
from jax.experimental.pallas import tpu_sc as plsc

_C = 21
_BATCH = 8192
_NW = 32
_IJS = 60
_NCHUNK = _IJS // 4
_PW = 8 * 1024
_TW = 8 * 512
_SCALE = 6.5131 / 40.0
_LN2 = 0.6931471805599453


def _sc_loss_partials(pred_lin, tgt_lin):
  mesh = plsc.VectorSubcoreMesh(core_axis_name="c", subcore_axis_name="s")

  @functools.partial(
      pl.kernel,
      out_type=jax.ShapeDtypeStruct((_NW * 80,), jnp.float32),
      mesh=mesh,
      compiler_params=pltpu.CompilerParams(needs_layout_passes=False),
      scratch_types=[
          pltpu.VMEM((2 * 3 * _PW,), jnp.float32),
          pltpu.VMEM((2 * _TW,), jnp.float32),
          pltpu.VMEM((80,), jnp.float32),
          pltpu.SemaphoreType.DMA,
          pltpu.SemaphoreType.DMA,
      ],
  )
  def k(pred_hbm, tgt_hbm, out_hbm, pred_buf, tgt_buf, acc_buf, sem0, sem1):
    wid = lax.axis_index("s") * 2 + lax.axis_index("c")

    zero = jnp.zeros((16,), jnp.float32)
    one = jnp.ones((16,), jnp.float32)
    ninf = jnp.full((16,), -jnp.inf, jnp.float32)

    def sigmoid(x):
      return 1.0 / (1.0 + jnp.exp(-x))

    def log_pos(u):
      bits = lax.bitcast_convert_type(u, jnp.int32)
      ex = lax.shift_right_logical(bits, 23) - 127
      m = lax.bitcast_convert_type(
          lax.bitwise_or(lax.bitwise_and(bits, 0x007FFFFF), 0x3F800000),
          jnp.float32)
      s = (m - 1.0) / (m + 1.0)
      t = s * s
      poly = 1.0 + t * (1.0 / 3.0 + t * (0.2 + t * (1.0 / 7.0 + t * (1.0 / 9.0))))
      lg = ex.astype(jnp.float32) * _LN2 + 2.0 * s * poly
      return jnp.where(u > 0, lg, ninf)

    def make_group_body(slot):
     def group_body(g, accs):
      xy, wh, obj, noobj, cls = accs
      u = lax.shift_right_logical(g, 3)
      sub = lax.bitwise_and(g, 7)
      q = u * 1024 + sub * 16 + slot * (3 * _PW)
      r = u * 512 + sub * 16 + slot * _TW

      t0 = tgt_buf[pl.ds(r, 16)]
      t1 = tgt_buf[pl.ds(r + 128, 16)]
      t2 = tgt_buf[pl.ds(r + 256, 16)]
      t3 = tgt_buf[pl.ds(r + 384, 16)]
      p0 = pred_buf[pl.ds(q, 16)]
      p1 = pred_buf[pl.ds(q + 128, 16)]
      p2 = pred_buf[pl.ds(q + 256, 16)]

      cm = t2 > 0.0
      d2 = sigmoid(p2) - t2
      s2 = d2 * d2
      o = jnp.where(cm, s2, zero)
      obj = obj + o
      noobj = noobj + (s2 - o)
      d0 = sigmoid(p0) - t0
      xy = xy + jnp.where(cm, d0 * d0, zero)
      d1 = p1 - log_pos(t1 * (1.0 / _SCALE))
      wh = wh + jnp.where(cm, d1 * d1, zero)

      num = zero
      den = zero
      for c in range(_C):
        cc = 3 + c
        off = (cc // 8) * _PW + (cc % 8) * 128
        e = jnp.exp(pred_buf[pl.ds(q + off, 16)])
        den = den + e
        num = num + e * (1.0 + 0.5 * c)
      diff = (10.0 * (num + den)) / (den * (t3 + 1.0)) - 10.0
      ad = jnp.abs(diff)
      sl1 = jnp.where(ad < 1.0, 0.5 * diff * diff, ad - 0.5)
      cls = cls + jnp.where(jnp.logical_and(cm, t3 > 0.0), sl1, zero)
      return (xy, wh, obj, noobj, cls)
     return group_body

    group_bodies = (make_group_body(0), make_group_body(1))
    sems = (sem0, sem1)

    def chunk_copies(ck, slot):
      gc = wid * _NCHUNK + ck
      ij = lax.shift_right_logical(gc, 3)
      bt0 = lax.bitwise_and(gc, 7) * 8
      pbase = (ij * 192 + bt0) * 1024
      ds = [
          pltpu.make_async_copy(
              pred_hbm.at[pl.ds(pbase + ct * 64 * 1024, _PW)],
              pred_buf.at[pl.ds(slot * 3 * _PW + ct * _PW, _PW)],
              sems[slot])
          for ct in range(3)
      ]
      ds.append(pltpu.make_async_copy(
          tgt_hbm.at[pl.ds((ij * 64 + bt0) * 512, _TW)],
          tgt_buf.at[pl.ds(slot * _TW, _TW)],
          sems[slot]))
      return ds

    def start_chunk(ck, slot):
      for d in chunk_copies(ck, slot):
        d.start()

    def wait_chunk(ck, slot):
      for d in chunk_copies(ck, slot):
        d.wait()

    def pair_body(m, accs):
      c0 = 2 * m
      start_chunk(c0 + 1, 1)
      wait_chunk(c0, 0)
      accs = lax.fori_loop(0, 64, group_bodies[0], accs)
      start_chunk(c0 + 2, 0)
      wait_chunk(c0 + 1, 1)
      accs = lax.fori_loop(0, 64, group_bodies[1], accs)
      return accs

    accs = (zero, zero, zero, zero, zero)
    start_chunk(0, 0)
    accs = lax.fori_loop(0, (_NCHUNK - 1) // 2, pair_body, accs)
    wait_chunk(_NCHUNK - 1, 0)
    accs = lax.fori_loop(0, 64, group_bodies[0], accs)
    xy, wh, obj, noobj, cls = accs
    acc_buf[pl.ds(0, 16)] = xy
    acc_buf[pl.ds(16, 16)] = wh
    acc_buf[pl.ds(32, 16)] = obj
    acc_buf[pl.ds(48, 16)] = noobj
    acc_buf[pl.ds(64, 16)] = cls
    pltpu.sync_copy(acc_buf, out_hbm.at[pl.ds(wid * 80, 80)])

  return k(pred_lin, tgt_lin)


def _tc_loss_partials(pred5, tgt4):
  nij = 100 - _IJS

  def body(pred_ref, tgt_ref, out_ref):
    ci = lax.broadcasted_iota(jnp.int32, (1, 8, 1), 1).astype(jnp.float32)
    w1s, wms = [], []
    for ct in range(3):
      ch = ci + float(ct * 8)
      w1 = (ch >= 3.0).astype(jnp.float32)
      w1s.append(w1)
      wms.append(w1 * (0.5 * ch - 0.5))
    zero = jnp.zeros((8, 128), jnp.float32)
    accs = [zero] * 5

    for g in range(8):
      tg = tgt_ref[0, pl.ds(8 * g, 8)]
      t0 = tg[:, 0, :]
      t1 = tg[:, 1, :]
      t2 = tg[:, 2, :]
      t3 = tg[:, 3, :]
      x0 = pred_ref[0, 0, pl.ds(8 * g, 8)]
      x1 = pred_ref[0, 1, pl.ds(8 * g, 8)]
      x2 = pred_ref[0, 2, pl.ds(8 * g, 8)]
      p0 = x0[:, 0, :]
      p1 = x0[:, 1, :]
      p2 = x0[:, 2, :]

      cm = t2 > 0.0
      d2 = 1.0 / (1.0 + jnp.exp(-p2)) - t2
      s2 = d2 * d2
      o = jnp.where(cm, s2, zero)
      obj = o
      noobj = s2 - o
      d0 = 1.0 / (1.0 + jnp.exp(-p0)) - t0
      xy = jnp.where(cm, d0 * d0, zero)
      d1 = p1 - jnp.log(t1 * (1.0 / _SCALE))
      wh = jnp.where(cm, d1 * d1, zero)

      e0 = jnp.exp(x0)
      dacc = e0 * w1s[0]
      nacc = e0 * wms[0]
      e1 = jnp.exp(x1)
      dacc = dacc + e1
      nacc = nacc + e1 * wms[1]
      e2 = jnp.exp(x2)
      dacc = dacc + e2
      nacc = nacc + e2 * wms[2]
      den = dacc.sum(axis=1)
      num = nacc.sum(axis=1)
      diff = (10.0 * (num + den)) / (den * (t3 + 1.0)) - 10.0
      ad = jnp.abs(diff)
      sl1 = jnp.where(ad < 1.0, 0.5 * diff * diff, ad - 0.5)
      cls = jnp.where(jnp.logical_and(cm, t3 > 0.0), sl1, zero)

      for a, v in enumerate((xy, wh, obj, noobj, cls)):
        accs[a] = accs[a] + v

    acc = jnp.concatenate(accs, axis=0)
    @pl.when(pl.program_id(0) == 0)
    def _():
      out_ref[...] = acc
    @pl.when(pl.program_id(0) > 0)
    def _():
      out_ref[...] += acc

  return pl.pallas_call(
      body,
      grid=(nij,),
      in_specs=[
          pl.BlockSpec((1, 3, 64, 8, 128), lambda i: (_IJS + i, 0, 0, 0, 0)),
          pl.BlockSpec((1, 64, 4, 128), lambda i: (_IJS + i, 0, 0, 0)),
      ],
      out_specs=pl.BlockSpec((40, 128), lambda i: (0, 0)),
      out_shape=jax.ShapeDtypeStruct((40, 128), jnp.float32),
      compiler_params=pltpu.CompilerParams(
          dimension_semantics=("arbitrary",)),
  )(pred5, tgt4)


def _linear_view_pred(p):
  return (p.transpose(1, 2, 3, 0)
           .reshape(10, 10, 3, 8, 64, 128)
           .transpose(0, 1, 2, 4, 3, 5)
           .reshape(-1))


def _linear_view_tgt(t):
  return (t.transpose(1, 2, 3, 0)
           .reshape(10, 10, 1, 4, 64, 128)
           .transpose(0, 1, 2, 4, 3, 5)
           .reshape(-1))


def kernel(pred_tensor, target_tensor):
  pred_lin = _linear_view_pred(pred_tensor)
  tgt_lin = _linear_view_tgt(target_tensor)
  tc_parts = _tc_loss_partials(pred_lin.reshape(100, 3, 64, 8, 128),
                               tgt_lin.reshape(100, 64, 4, 128))
  parts = _sc_loss_partials(pred_lin, tgt_lin)
  sums = (parts.reshape(_NW, 5, 16).sum(axis=(0, 2))
          + tc_parts.reshape(5, 1024).sum(axis=1))
  xy, wh, obj, noobj, cls = sums[0], sums[1], sums[2], sums[3], sums[4]
  loss = 10.0 * (xy + wh) + obj + noobj + 0.5 * cls
  bs = jnp.float32(_BATCH)
  return (xy / bs, wh / bs, obj / bs, noobj / bs, cls / bs, loss / bs)


# speedup vs baseline: 1.1050x; 1.0124x over previous
import functools

import jax
import jax.numpy as jnp
from jax import lax
from jax.experimental import pallas as pl
from jax.experimental.pallas import tpu as pltpu
from jax.experimental.pallas import tpu_sc as plsc

_C = 21
_BATCH = 8192
_NW = 32
_IJS = 64
_NCHUNK = _IJS // 4
_PW = 8 * 1024
_TW = 8 * 512
_SCALE = 6.5131 / 40.0
_LN2 = 0.6931471805599453


def _sc_loss_partials(pred_lin, tgt_lin):
  mesh = plsc.VectorSubcoreMesh(core_axis_name="c", subcore_axis_name="s")

  @functools.partial(
      pl.kernel,
      out_type=jax.ShapeDtypeStruct((_NW * 80,), jnp.float32),
      mesh=mesh,
      compiler_params=pltpu.CompilerParams(needs_layout_passes=False),
      scratch_types=[
          pltpu.VMEM((2 * 3 * _PW,), jnp.float32),
          pltpu.VMEM((2 * _TW,), jnp.float32),
          pltpu.VMEM((80,), jnp.float32),
          pltpu.SemaphoreType.DMA,
          pltpu.SemaphoreType.DMA,
      ],
  )
  def k(pred_hbm, tgt_hbm, out_hbm, pred_buf, tgt_buf, acc_buf, sem0, sem1):
    wid = lax.axis_index("s") * 2 + lax.axis_index("c")

    zero = jnp.zeros((16,), jnp.float32)
    one = jnp.ones((16,), jnp.float32)
    ninf = jnp.full((16,), -jnp.inf, jnp.float32)

    def sigmoid(x):
      return 1.0 / (1.0 + jnp.exp(-x))

    def log_pos(u):
      bits = lax.bitcast_convert_type(u, jnp.int32)
      ex = lax.shift_right_logical(bits, 23) - 127
      m = lax.bitcast_convert_type(
          lax.bitwise_or(lax.bitwise_and(bits, 0x007FFFFF), 0x3F800000),
          jnp.float32)
      s = (m - 1.0) / (m + 1.0)
      t = s * s
      poly = 1.0 + t * (1.0 / 3.0 + t * (0.2 + t * (1.0 / 7.0 + t * (1.0 / 9.0))))
      lg = ex.astype(jnp.float32) * _LN2 + 2.0 * s * poly
      return jnp.where(u > 0, lg, ninf)

    def make_group_body(slot):
     def group_body(g, accs):
      xy, wh, obj, noobj, cls = accs
      u = lax.shift_right_logical(g, 3)
      sub = lax.bitwise_and(g, 7)
      q = u * 1024 + sub * 16 + slot * (3 * _PW)
      r = u * 512 + sub * 16 + slot * _TW

      t0 = tgt_buf[pl.ds(r, 16)]
      t1 = tgt_buf[pl.ds(r + 128, 16)]
      t2 = tgt_buf[pl.ds(r + 256, 16)]
      t3 = tgt_buf[pl.ds(r + 384, 16)]
      p0 = pred_buf[pl.ds(q, 16)]
      p1 = pred_buf[pl.ds(q + 128, 16)]
      p2 = pred_buf[pl.ds(q + 256, 16)]

      cm = t2 > 0.0
      d2 = sigmoid(p2) - t2
      s2 = d2 * d2
      o = jnp.where(cm, s2, zero)
      obj = obj + o
      noobj = noobj + (s2 - o)
      d0 = sigmoid(p0) - t0
      xy = xy + jnp.where(cm, d0 * d0, zero)
      d1 = p1 - log_pos(t1 * (1.0 / _SCALE))
      wh = wh + jnp.where(cm, d1 * d1, zero)

      num = zero
      den = zero
      for c in range(_C):
        cc = 3 + c
        off = (cc // 8) * _PW + (cc % 8) * 128
        e = jnp.exp(pred_buf[pl.ds(q + off, 16)])
        den = den + e
        num = num + e * (1.0 + 0.5 * c)
      diff = (10.0 * (num + den)) / (den * (t3 + 1.0)) - 10.0
      ad = jnp.abs(diff)
      sl1 = jnp.where(ad < 1.0, 0.5 * diff * diff, ad - 0.5)
      cls = cls + jnp.where(jnp.logical_and(cm, t3 > 0.0), sl1, zero)
      return (xy, wh, obj, noobj, cls)
     return group_body

    group_bodies = (make_group_body(0), make_group_body(1))
    sems = (sem0, sem1)

    def chunk_copies(ck, slot):
      gc = wid * _NCHUNK + ck
      ij = lax.shift_right_logical(gc, 3)
      bt0 = lax.bitwise_and(gc, 7) * 8
      pbase = (ij * 192 + bt0) * 1024
      ds = [
          pltpu.make_async_copy(
              pred_hbm.at[pl.ds(pbase + ct * 64 * 1024, _PW)],
              pred_buf.at[pl.ds(slot * 3 * _PW + ct * _PW, _PW)],
              sems[slot])
          for ct in range(3)
      ]
      ds.append(pltpu.make_async_copy(
          tgt_hbm.at[pl.ds((ij * 64 + bt0) * 512, _TW)],
          tgt_buf.at[pl.ds(slot * _TW, _TW)],
          sems[slot]))
      return ds

    def start_chunk(ck, slot):
      for d in chunk_copies(ck, slot):
        d.start()

    def wait_chunk(ck, slot):
      for d in chunk_copies(ck, slot):
        d.wait()

    def pair_body(m, accs):
      c0 = 2 * m
      start_chunk(c0 + 1, 1)
      wait_chunk(c0, 0)
      accs = lax.fori_loop(0, 64, group_bodies[0], accs)
      start_chunk(c0 + 2, 0)
      wait_chunk(c0 + 1, 1)
      accs = lax.fori_loop(0, 64, group_bodies[1], accs)
      return accs

    accs = (zero, zero, zero, zero, zero)
    start_chunk(0, 0)
    accs = lax.fori_loop(0, (_NCHUNK - 1) // 2, pair_body, accs)
    if _NCHUNK % 2 == 0:
      start_chunk(_NCHUNK - 1, 1)
      wait_chunk(_NCHUNK - 2, 0)
      accs = lax.fori_loop(0, 64, group_bodies[0], accs)
      wait_chunk(_NCHUNK - 1, 1)
      accs = lax.fori_loop(0, 64, group_bodies[1], accs)
    else:
      wait_chunk(_NCHUNK - 1, 0)
      accs = lax.fori_loop(0, 64, group_bodies[0], accs)
    xy, wh, obj, noobj, cls = accs
    acc_buf[pl.ds(0, 16)] = xy
    acc_buf[pl.ds(16, 16)] = wh
    acc_buf[pl.ds(32, 16)] = obj
    acc_buf[pl.ds(48, 16)] = noobj
    acc_buf[pl.ds(64, 16)] = cls
    pltpu.sync_copy(acc_buf, out_hbm.at[pl.ds(wid * 80, 80)])

  return k(pred_lin, tgt_lin)


def _tc_loss_partials(pred5, tgt4):
  nij = 100 - _IJS

  def body(pred_ref, tgt_ref, out_ref):
    ci = lax.broadcasted_iota(jnp.int32, (1, 8, 1), 1).astype(jnp.float32)
    w1s, wms = [], []
    for ct in range(3):
      ch = ci + float(ct * 8)
      w1 = (ch >= 3.0).astype(jnp.float32)
      w1s.append(w1)
      wms.append(w1 * (0.5 * ch - 0.5))
    zero = jnp.zeros((8, 128), jnp.float32)
    accs = [zero] * 5

    for g in range(8):
      tg = tgt_ref[0, pl.ds(8 * g, 8)]
      t0 = tg[:, 0, :]
      t1 = tg[:, 1, :]
      t2 = tg[:, 2, :]
      t3 = tg[:, 3, :]
      x0 = pred_ref[0, 0, pl.ds(8 * g, 8)]
      x1 = pred_ref[0, 1, pl.ds(8 * g, 8)]
      x2 = pred_ref[0, 2, pl.ds(8 * g, 8)]
      p0 = x0[:, 0, :]
      p1 = x0[:, 1, :]
      p2 = x0[:, 2, :]

      cm = t2 > 0.0
      d2 = 1.0 / (1.0 + jnp.exp(-p2)) - t2
      s2 = d2 * d2
      o = jnp.where(cm, s2, zero)
      obj = o
      noobj = s2 - o
      d0 = 1.0 / (1.0 + jnp.exp(-p0)) - t0
      xy = jnp.where(cm, d0 * d0, zero)
      d1 = p1 - jnp.log(t1 * (1.0 / _SCALE))
      wh = jnp.where(cm, d1 * d1, zero)

      e0 = jnp.exp(x0)
      dacc = e0 * w1s[0]
      nacc = e0 * wms[0]
      e1 = jnp.exp(x1)
      dacc = dacc + e1
      nacc = nacc + e1 * wms[1]
      e2 = jnp.exp(x2)
      dacc = dacc + e2
      nacc = nacc + e2 * wms[2]
      den = dacc.sum(axis=1)
      num = nacc.sum(axis=1)
      diff = (10.0 * (num + den)) / (den * (t3 + 1.0)) - 10.0
      ad = jnp.abs(diff)
      sl1 = jnp.where(ad < 1.0, 0.5 * diff * diff, ad - 0.5)
      cls = jnp.where(jnp.logical_and(cm, t3 > 0.0), sl1, zero)

      for a, v in enumerate((xy, wh, obj, noobj, cls)):
        accs[a] = accs[a] + v

    acc = jnp.concatenate(accs, axis=0)
    @pl.when(pl.program_id(0) == 0)
    def _():
      out_ref[...] = acc
    @pl.when(pl.program_id(0) > 0)
    def _():
      out_ref[...] += acc

  return pl.pallas_call(
      body,
      grid=(nij,),
      in_specs=[
          pl.BlockSpec((1, 3, 64, 8, 128), lambda i: (_IJS + i, 0, 0, 0, 0)),
          pl.BlockSpec((1, 64, 4, 128), lambda i: (_IJS + i, 0, 0, 0)),
      ],
      out_specs=pl.BlockSpec((40, 128), lambda i: (0, 0)),
      out_shape=jax.ShapeDtypeStruct((40, 128), jnp.float32),
      compiler_params=pltpu.CompilerParams(
          dimension_semantics=("arbitrary",)),
  )(pred5, tgt4)


def _linear_view_pred(p):
  return (p.transpose(1, 2, 3, 0)
           .reshape(10, 10, 3, 8, 64, 128)
           .transpose(0, 1, 2, 4, 3, 5)
           .reshape(-1))


def _linear_view_tgt(t):
  return (t.transpose(1, 2, 3, 0)
           .reshape(10, 10, 1, 4, 64, 128)
           .transpose(0, 1, 2, 4, 3, 5)
           .reshape(-1))


def kernel(pred_tensor, target_tensor):
  pred_lin = _linear_view_pred(pred_tensor)
  tgt_lin = _linear_view_tgt(target_tensor)
  tc_parts = _tc_loss_partials(pred_lin.reshape(100, 3, 64, 8, 128),
                               tgt_lin.reshape(100, 64, 4, 128))
  parts = _sc_loss_partials(pred_lin, tgt_lin)
  sums = (parts.reshape(_NW, 5, 16).sum(axis=(0, 2))
          + tc_parts.reshape(5, 1024).sum(axis=1))
  xy, wh, obj, noobj, cls = sums[0], sums[1], sums[2], sums[3], sums[4]
  loss = 10.0 * (xy + wh) + obj + noobj + 0.5 * cls
  bs = jnp.float32(_BATCH)
  return (xy / bs, wh / bs, obj / bs, noobj / bs, cls / bs, loss / bs)
